# baseline (speedup 1.0000x reference)
import jax
import jax.numpy as jnp
from jax import lax
from jax.experimental import pallas as pl
from jax.experimental.pallas import tpu as pltpu
from jax.experimental.pallas import tpu_sc as plsc

_BATCH = 16384
_HID = 16
_NW = 32
_PER_W = _BATCH // _NW
_CH = 8
_NPAIR = _PER_W // (2 * _CH)


def _mf_body(u_idx_hbm, i_idx_hbm, u_t3_hbm, i_t3_hbm, out_hbm,
             uidx_v, iidx_v, ub_a, ib_a, ub_b, ib_b, prods_v, out_v,
             sem_a, sem_b):
    nc = 2
    wid = lax.axis_index("s") * nc + lax.axis_index("c")

    pltpu.sync_copy(u_idx_hbm.at[wid], uidx_v)
    pltpu.sync_copy(i_idx_hbm.at[wid], iidx_v)

    lane = lax.iota(jnp.int32, 16)
    i_vec = lane // 8
    d_vec = lane % 8

    def fire(j, half, ub, ib, sem):
        iu = uidx_v[0, pl.ds(j * 16, 16)]
        ii = iidx_v[0, pl.ds(j * 16, 16)]
        for k in range(_CH):
            bu = pl.multiple_of((iu[half * _CH + k] // 128) * 128, 128)
            bi = pl.multiple_of((ii[half * _CH + k] // 128) * 128, 128)
            dst = pl.ds(k * 128, 128)
            pltpu.async_copy(u_t3_hbm.at[:, :, pl.ds(bu, 128)],
                             ub.at[:, :, dst], sem)
            pltpu.async_copy(i_t3_hbm.at[:, :, pl.ds(bi, 128)],
                             ib.at[:, :, dst], sem)

    def drain(ub, ib, sem):
        dummy = u_t3_hbm.at[:, :, pl.ds(0, _CH * 128)]
        pltpu.make_async_copy(dummy, ub, sem).wait()
        pltpu.make_async_copy(dummy, ib, sem).wait()

    def compute_half(j, ub, ib, half):
        iu = uidx_v[0, pl.ds(j * 16, 16)]
        ii = iidx_v[0, pl.ds(j * 16, 16)]
        for k in range(_CH):
            cu = jnp.full((16,), iu[half * _CH + k] % 128 + k * 128, jnp.int32)
            ci = jnp.full((16,), ii[half * _CH + k] % 128 + k * 128, jnp.int32)
            uv = plsc.load_gather(ub, [i_vec, d_vec, cu])
            iv = plsc.load_gather(ib, [i_vec, d_vec, ci])
            prods_v[half * _CH + k, pl.ds(0, 16)] = uv * iv

    fire(0, 0, ub_a, ib_a, sem_a)

    def body(j, carry):
        fire(j, 1, ub_b, ib_b, sem_b)
        drain(ub_a, ib_a, sem_a)
        compute_half(j, ub_a, ib_a, 0)

        @pl.when(j + 1 < _NPAIR)
        def _():
            fire(j + 1, 0, ub_a, ib_a, sem_a)

        drain(ub_b, ib_b, sem_b)
        compute_half(j, ub_b, ib_b, 1)

        acc = jnp.zeros((16,), jnp.float32)
        for d in range(_HID):
            dd = jnp.full((16,), d, jnp.int32)
            acc = acc + plsc.load_gather(prods_v, [lane, dd])
        out_v[0, pl.ds(j * 16, 16)] = acc
        return carry

    lax.fori_loop(0, _NPAIR, body, 0)

    pltpu.sync_copy(out_v, out_hbm.at[wid])


def kernel(user_indices, item_indices, embed_user_w, embed_item_w):
    u_idx = user_indices.astype(jnp.int32).reshape(_NW, 1, _PER_W)
    i_idx = item_indices.astype(jnp.int32).reshape(_NW, 1, _PER_W)
    u_t3 = embed_user_w.T.reshape(2, 8, 1000001)
    i_t3 = embed_item_w.T.reshape(2, 8, 1000001)

    mesh = plsc.VectorSubcoreMesh(core_axis_name="c", subcore_axis_name="s")
    run = pl.kernel(
        _mf_body, mesh=mesh,
        out_type=jax.ShapeDtypeStruct((_NW, 1, _PER_W), jnp.float32),
        scratch_types=[
            pltpu.VMEM((1, _PER_W), jnp.int32),
            pltpu.VMEM((1, _PER_W), jnp.int32),
            pltpu.VMEM((2, 8, _CH * 128), jnp.float32),
            pltpu.VMEM((2, 8, _CH * 128), jnp.float32),
            pltpu.VMEM((2, 8, _CH * 128), jnp.float32),
            pltpu.VMEM((2, 8, _CH * 128), jnp.float32),
            pltpu.VMEM((2 * _CH, 128), jnp.float32),
            pltpu.VMEM((1, _PER_W), jnp.float32),
            pltpu.SemaphoreType.DMA,
            pltpu.SemaphoreType.DMA,
        ],
        compiler_params=pltpu.CompilerParams(needs_layout_passes=False),
    )
    out = run(u_idx, i_idx, u_t3, i_t3)
    return out.reshape(_BATCH)
